# baseline (speedup 1.0000x reference)
import functools

import jax
import jax.numpy as jnp
from jax import lax
from jax.experimental import pallas as pl
from jax.experimental.pallas import tpu as pltpu
from jax.experimental.pallas import tpu_sc as plsc

_USERS_BUCKETS = 1000000
_MOVIES_BUCKETS = 100000
_D = 32
_B = 16384
_NW = 32
_BPW = _B // _NW
_L = 16
_BLK = 128
_NBLK = _BPW // _BLK

_W_U = (_USERS_BUCKETS + 127) // 128 * 128
_S_U = 8 * _W_U
_TOT_U = _D * _W_U


def _vec_mod(v, n):
  q = (v.astype(jnp.float32) * (1.0 / n)).astype(jnp.int32)
  r = v - q * n
  r = jnp.where(r >= n, r - n, r)
  r = jnp.where(r < 0, r + n, r)
  return r


def _physical_flat(table, w):
  t = table.T
  rows = t.shape[1]
  t = jnp.pad(t, ((0, 0), (0, w - rows)))
  t = t.reshape(4, 8, w // 128, 128).transpose(0, 2, 1, 3)
  return t.reshape(-1)


def _body(uids_hbm, mids_hbm, utab_hbm, mtab_hbm, out_hbm,
          uids_v, mids_v, idxu_v, idxm_v, uvals_v, mvals_v, out_v, sem):
  wid = lax.axis_index("s") * 2 + lax.axis_index("c")
  base = wid * _BPW

  cps = [pltpu.async_copy(uids_hbm.at[pl.ds(base, _BPW)], uids_v, sem),
         pltpu.async_copy(mids_hbm.at[pl.ds(base, _BPW)], mids_v, sem)]
  for cp in cps:
    cp.wait()

  for l in range(_BPW // _L):
    sl = pl.ds(l * _L, _L)
    ru = _vec_mod(uids_v[sl], _USERS_BUCKETS)
    idxu_v[sl] = ((ru >> 7) << 10) + (ru & 127)
    idxm_v[sl] = _vec_mod(mids_v[sl], _MOVIES_BUCKETS)

  copies = []
  for kb in range(_NBLK):
    isl = pl.ds(kb * _BLK, _BLK)
    copies.append(pltpu.async_copy(
        mtab_hbm.at[idxm_v.at[isl]],
        mvals_v.at[pl.ds(kb * _BLK, _BLK)], sem))
    for j in range(_D):
      offu = (j // 8) * _S_U + (j % 8) * 128
      vsl = pl.ds((kb * _D + j) * _BLK, _BLK)
      copies.append(pltpu.async_copy(
          utab_hbm.at[pl.ds(offu, _TOT_U - offu)].at[idxu_v.at[isl]],
          uvals_v.at[vsl], sem))
  for cp in copies:
    cp.wait()

  lane = lax.iota(jnp.int32, _L)

  def block_body(kb, carry):
    vbase = kb * _D * _BLK
    for l in range(_BLK // _L):
      ridx = kb * _BLK + l * _L + lane
      acc = jnp.zeros((_L,), jnp.float32)
      for j in range(_D):
        sl = pl.ds(vbase + j * _BLK + l * _L, _L)
        mm = plsc.load_gather(mvals_v, [ridx, jnp.full((_L,), j, jnp.int32)])
        acc = acc + uvals_v[sl] * mm
      out_v[pl.ds(kb * _BLK + l * _L, _L)] = 1.0 / (1.0 + jnp.exp(-acc))
    return carry

  lax.fori_loop(0, _NBLK, block_body, 0)

  pltpu.sync_copy(out_v, out_hbm.at[pl.ds(base, _BPW)])


def kernel(users_ids, movies_ids, user_table, movie_table):
  mesh = plsc.VectorSubcoreMesh(core_axis_name="c", subcore_axis_name="s")
  run = functools.partial(
      pl.kernel,
      mesh=mesh,
      compiler_params=pltpu.CompilerParams(
          needs_layout_passes=False, use_tc_tiling_on_sc=False,
          disable_bounds_checks=True),
      out_type=jax.ShapeDtypeStruct((_B,), jnp.float32),
      scratch_types=[
          pltpu.VMEM((_BPW,), jnp.int32),
          pltpu.VMEM((_BPW,), jnp.int32),
          pltpu.VMEM((_BPW,), jnp.int32),
          pltpu.VMEM((_BPW,), jnp.int32),
          pltpu.VMEM((_D * _BPW,), jnp.float32),
          pltpu.VMEM((_BPW, _D), jnp.float32),
          pltpu.VMEM((_BPW,), jnp.float32),
          pltpu.SemaphoreType.DMA,
      ],
  )(_body)
  return run(users_ids, movies_ids,
             _physical_flat(user_table, _W_U),
             movie_table)


# speedup vs baseline: 1.1474x; 1.1474x over previous
import functools

import jax
import jax.numpy as jnp
from jax import lax
from jax.experimental import pallas as pl
from jax.experimental.pallas import tpu as pltpu
from jax.experimental.pallas import tpu_sc as plsc

_USERS_BUCKETS = 1000000
_MOVIES_BUCKETS = 100000
_D = 32
_B = 16384
_NW = 32
_BPW = _B // _NW
_L = 16
_BLK = 128
_NBLK = _BPW // _BLK

_W_U = (_USERS_BUCKETS + 127) // 128 * 128
_W_M = (_MOVIES_BUCKETS + 127) // 128 * 128
_S_U = 8 * _W_U
_S_M = 8 * _W_M
_TOT_U = _D * _W_U
_TOT_M = _D * _W_M


def _vec_mod(v, n):
  q = (v.astype(jnp.float32) * (1.0 / n)).astype(jnp.int32)
  r = v - q * n
  r = jnp.where(r >= n, r - n, r)
  r = jnp.where(r < 0, r + n, r)
  return r


def _physical_flat(table, w):
  t = table.T
  rows = t.shape[1]
  t = jnp.pad(t, ((0, 0), (0, w - rows)))
  t = t.reshape(4, 8, w // 128, 128).transpose(0, 2, 1, 3)
  return t.reshape(-1)


def _body(uids_hbm, mids_hbm, utab_hbm, mtab_hbm, out_hbm,
          uids_v, mids_v, idxu_v, idxm_v, uvals_v, mvals_v, out_v, sem):
  wid = lax.axis_index("s") * 2 + lax.axis_index("c")
  base = wid * _BPW

  cp_u = pltpu.async_copy(uids_hbm.at[pl.ds(base, _BPW)], uids_v, sem)
  cp_m = pltpu.async_copy(mids_hbm.at[pl.ds(base, _BPW)], mids_v, sem)
  cp_u.wait()
  cp_m.wait()

  for l in range(_BPW // _L):
    sl = pl.ds(l * _L, _L)
    ru = _vec_mod(uids_v[sl], _USERS_BUCKETS)
    rm = _vec_mod(mids_v[sl], _MOVIES_BUCKETS)
    idxu_v[sl] = ((ru >> 7) << 10) + (ru & 127)
    idxm_v[sl] = ((rm >> 7) << 10) + (rm & 127)

  copies = []
  for kb in range(_NBLK):
    isl = pl.ds(kb * _BLK, _BLK)
    for j in range(_D):
      offu = (j // 8) * _S_U + (j % 8) * 128
      offm = (j // 8) * _S_M + (j % 8) * 128
      vsl = pl.ds((kb * _D + j) * _BLK, _BLK)
      copies.append(pltpu.async_copy(
          utab_hbm.at[pl.ds(offu, _TOT_U - offu)].at[idxu_v.at[isl]],
          uvals_v.at[vsl], sem))
      copies.append(pltpu.async_copy(
          mtab_hbm.at[pl.ds(offm, _TOT_M - offm)].at[idxm_v.at[isl]],
          mvals_v.at[vsl], sem))
  for cp in copies:
    cp.wait()

  def block_body(kb, carry):
    vbase = kb * _D * _BLK
    for l in range(_BLK // _L):
      acc = jnp.zeros((_L,), jnp.float32)
      for j in range(_D):
        sl = pl.ds(vbase + j * _BLK + l * _L, _L)
        acc = acc + uvals_v[sl] * mvals_v[sl]
      out_v[pl.ds(kb * _BLK + l * _L, _L)] = 1.0 / (1.0 + jnp.exp(-acc))
    return carry

  lax.fori_loop(0, _NBLK, block_body, 0)

  pltpu.sync_copy(out_v, out_hbm.at[pl.ds(base, _BPW)])


def kernel(users_ids, movies_ids, user_table, movie_table):
  mesh = plsc.VectorSubcoreMesh(core_axis_name="c", subcore_axis_name="s")
  run = functools.partial(
      pl.kernel,
      mesh=mesh,
      compiler_params=pltpu.CompilerParams(
          needs_layout_passes=False, use_tc_tiling_on_sc=False,
          disable_bounds_checks=True),
      out_type=jax.ShapeDtypeStruct((_B,), jnp.float32),
      scratch_types=[
          pltpu.VMEM((_BPW,), jnp.int32),
          pltpu.VMEM((_BPW,), jnp.int32),
          pltpu.VMEM((_BPW,), jnp.int32),
          pltpu.VMEM((_BPW,), jnp.int32),
          pltpu.VMEM((_D * _BPW,), jnp.float32),
          pltpu.VMEM((_D * _BPW,), jnp.float32),
          pltpu.VMEM((_BPW,), jnp.float32),
          pltpu.SemaphoreType.DMA,
      ],
  )(_body)
  return run(users_ids, movies_ids,
             _physical_flat(user_table, _W_U),
             _physical_flat(movie_table, _W_M))
